# baseline (speedup 1.0000x reference)
import functools

import jax
import jax.numpy as jnp
from jax import lax
from jax.experimental import pallas as pl
from jax.experimental.pallas import tpu as pltpu
from jax.experimental.pallas import tpu_sc as plsc

N = 10000
NP = 10240
E = 320000
EP = 327680
CHUNK = 128
NCHUNK = EP // (16 * CHUNK)
GROUP = 16
GSEG = 32
ROWS_PER_TILE = NP // 16

@functools.lru_cache(maxsize=1)
def _make_segsum():
    mesh = plsc.VectorSubcoreMesh(
        core_axis_name="c", subcore_axis_name="s", num_cores=2, num_subcores=16)

    @functools.partial(
        pl.kernel,
        out_type=jax.ShapeDtypeStruct((2, NP, 128), jnp.float32),
        mesh=mesh,
        scratch_types=[
            pltpu.VMEM((GSEG, CHUNK), jnp.int32),
            pltpu.VMEM((GSEG, CHUNK), jnp.int32),
            pltpu.VMEM((CHUNK, 128), jnp.float32),
            pltpu.VMEM((CHUNK, 128), jnp.float32),
            pltpu.VMEM_SHARED((NP, 128), jnp.float32),
            pltpu.SemaphoreType.DMA,
            pltpu.SemaphoreType.DMA,
            pltpu.SemaphoreType.DMA,
            pltpu.SemaphoreType.DMA,
        ],
    )
    def _segsum(h_flat, src2, dstp, zblk, agg_out,
                src_v, dst_v, rows_a, rows_b, agg_sh, ga, gb, sa, sb):
        c = lax.axis_index("c")
        s = lax.axis_index("s")
        pltpu.sync_copy(zblk, agg_sh.at[pl.ds(s * ROWS_PER_TILE, ROWS_PER_TILE)])
        plsc.subcore_barrier()

        def gather(j, buf, sem):
            pltpu.async_copy(h_flat.at[src_v.at[j]], buf, sem)

        def gwait(buf, sem):
            pltpu.make_async_copy(h_flat.at[src_v.at[0]], buf, sem).wait()

        def scat(j, buf, sem):
            pltpu.async_copy(buf, agg_sh.at[dst_v.at[j]], sem, add=True)

        def swait(buf, sem):
            pltpu.make_async_copy(buf, agg_sh.at[dst_v.at[0]], sem).wait()

        NPAIR = GSEG // 2

        def group(g, carry):
            pltpu.sync_copy(src2.at[c, s, pl.ds(g * GSEG, GSEG)], src_v)
            pltpu.sync_copy(dstp.at[s, pl.ds(g * GSEG, GSEG)], dst_v)
            gather(0, rows_a, ga)

            def pair(p, carry2):
                j0 = 2 * p
                gwait(rows_a, ga)

                @pl.when(p > 0)
                def _():
                    swait(rows_b, sb)

                gather(j0 + 1, rows_b, gb)
                scat(j0, rows_a, sa)
                gwait(rows_b, gb)
                swait(rows_a, sa)

                @pl.when(p < NPAIR - 1)
                def _():
                    gather(j0 + 2, rows_a, ga)

                scat(j0 + 1, rows_b, sb)
                return carry2

            carry = lax.fori_loop(0, NPAIR, pair, carry)
            swait(rows_b, sb)
            return carry

        lax.fori_loop(0, NCHUNK // GSEG, group, 0)
        plsc.subcore_barrier()
        pltpu.sync_copy(agg_sh.at[pl.ds(s * ROWS_PER_TILE, ROWS_PER_TILE)],
                        agg_out.at[c, pl.ds(s * ROWS_PER_TILE, ROWS_PER_TILE)])

    return _segsum


NCHUNK_X = EP // (2 * 16 * CHUNK)


@functools.lru_cache(maxsize=1)
def _make_segsum_x():
    mesh = plsc.VectorSubcoreMesh(
        core_axis_name="c", subcore_axis_name="s", num_cores=2, num_subcores=16)

    @functools.partial(
        pl.kernel,
        out_type=jax.ShapeDtypeStruct((2, NP, 128), jnp.float32),
        mesh=mesh,
        scratch_types=[
            pltpu.VMEM((GROUP, CHUNK), jnp.int32),
            pltpu.VMEM((GROUP, CHUNK), jnp.int32),
            pltpu.VMEM((CHUNK, 128), jnp.float32),
            pltpu.VMEM((CHUNK, 128), jnp.float32),
            pltpu.VMEM_SHARED((NP, 128), jnp.float32),
            pltpu.SemaphoreType.DMA,
            pltpu.SemaphoreType.DMA,
            pltpu.SemaphoreType.DMA,
            pltpu.SemaphoreType.DMA,
        ],
    )
    def _segsum_x(x_tab, srcx, dstx, zblk, acc_out,
                  src_v, dst_v, rows_a, rows_b, agg_sh, ga, gb, sa, sb):
        c = lax.axis_index("c")
        s = lax.axis_index("s")
        pltpu.sync_copy(zblk, agg_sh.at[pl.ds(s * ROWS_PER_TILE, ROWS_PER_TILE)])
        plsc.subcore_barrier()

        def gather(j, buf, sem):
            pltpu.async_copy(x_tab.at[src_v.at[j]], buf, sem)

        def gwait(buf, sem):
            pltpu.make_async_copy(x_tab.at[src_v.at[0]], buf, sem).wait()

        def scat(j, buf, sem):
            pltpu.async_copy(buf, agg_sh.at[dst_v.at[j]], sem, add=True)

        def swait(buf, sem):
            pltpu.make_async_copy(buf, agg_sh.at[dst_v.at[0]], sem).wait()

        NPAIR = GROUP // 2

        def group(g, carry):
            pltpu.sync_copy(srcx.at[c, s, pl.ds(g * GROUP, GROUP)], src_v)
            pltpu.sync_copy(dstx.at[c, s, pl.ds(g * GROUP, GROUP)], dst_v)
            gather(0, rows_a, ga)

            def pair(p, carry2):
                j0 = 2 * p
                gwait(rows_a, ga)

                @pl.when(p > 0)
                def _():
                    swait(rows_b, sb)

                gather(j0 + 1, rows_b, gb)
                scat(j0, rows_a, sa)
                gwait(rows_b, gb)
                swait(rows_a, sa)

                @pl.when(p < NPAIR - 1)
                def _():
                    gather(j0 + 2, rows_a, ga)

                scat(j0 + 1, rows_b, sb)
                return carry2

            carry = lax.fori_loop(0, NPAIR, pair, carry)
            swait(rows_b, sb)
            return carry

        lax.fori_loop(0, NCHUNK_X // GROUP, group, 0)
        plsc.subcore_barrier()
        pltpu.sync_copy(agg_sh.at[pl.ds(s * ROWS_PER_TILE, ROWS_PER_TILE)],
                        acc_out.at[c, pl.ds(s * ROWS_PER_TILE, ROWS_PER_TILE)])

    return _segsum_x


_BM = 512
_NBLK = NP // _BM


def _proj_body(x_ref, w_ref, out_ref):
    out_ref[0] = jnp.dot(x_ref[...], w_ref[...],
                         preferred_element_type=jnp.float32)


def _proj_in(x_pad, W_in):
    return pl.pallas_call(
        _proj_body,
        grid=(2, _NBLK),
        in_specs=[
            pl.BlockSpec((_BM, 128), lambda c, i: (i, 0)),
            pl.BlockSpec((128, 128), lambda c, i: (0, c)),
        ],
        out_specs=pl.BlockSpec((1, _BM, 128), lambda c, i: (c, i, 0)),
        out_shape=jax.ShapeDtypeStruct((2, NP, 128), jnp.float32),
    )(x_pad, W_in)


def _gin0_body(h_ref, a_ref, w1_ref, wx1_ref, b1_ref, w2_ref, b2_ref, out_ref):
    hf = jnp.concatenate([h_ref[0], h_ref[1]], axis=-1)
    ax = a_ref[0] + a_ref[1]
    u = jnp.maximum(
        jnp.dot(hf, w1_ref[...], preferred_element_type=jnp.float32)
        + jnp.dot(ax, wx1_ref[...], preferred_element_type=jnp.float32)
        + b1_ref[...], 0.0)
    h2 = jnp.dot(u, w2_ref[...], preferred_element_type=jnp.float32) + b2_ref[...]
    out_ref[0] = h2[:, :128]
    out_ref[1] = h2[:, 128:]


def _gin0_layer(h_cat, ax_pair, W1, Wx1, b1, W2, b2):
    full = lambda i: (0, 0)
    return pl.pallas_call(
        _gin0_body,
        grid=(_NBLK,),
        in_specs=[
            pl.BlockSpec((2, _BM, 128), lambda i: (0, i, 0)),
            pl.BlockSpec((2, _BM, 128), lambda i: (0, i, 0)),
            pl.BlockSpec((256, 256), full),
            pl.BlockSpec((128, 256), full),
            pl.BlockSpec((1, 256), full),
            pl.BlockSpec((256, 256), full),
            pl.BlockSpec((1, 256), full),
        ],
        out_specs=pl.BlockSpec((2, _BM, 128), lambda i: (0, i, 0)),
        out_shape=jax.ShapeDtypeStruct((2, NP, 128), jnp.float32),
    )(h_cat, ax_pair, W1, Wx1, b1, W2, b2)


def _gin_body(h_ref, a_ref, w1_ref, b1_ref, w2_ref, b2_ref, out_ref):
    hf = jnp.concatenate([h_ref[0] + a_ref[0], h_ref[1] + a_ref[1]], axis=-1)
    u = jnp.maximum(jnp.dot(hf, w1_ref[...],
                            preferred_element_type=jnp.float32) + b1_ref[...], 0.0)
    h2 = jnp.dot(u, w2_ref[...], preferred_element_type=jnp.float32) + b2_ref[...]
    out_ref[0] = h2[:, :128]
    out_ref[1] = h2[:, 128:]


def _gin_layer(h_cat, agg_cat, W1, b1, W2, b2):
    full = lambda i: (0, 0)
    return pl.pallas_call(
        _gin_body,
        grid=(_NBLK,),
        in_specs=[
            pl.BlockSpec((2, _BM, 128), lambda i: (0, i, 0)),
            pl.BlockSpec((2, _BM, 128), lambda i: (0, i, 0)),
            pl.BlockSpec((256, 256), full),
            pl.BlockSpec((1, 256), full),
            pl.BlockSpec((256, 256), full),
            pl.BlockSpec((1, 256), full),
        ],
        out_specs=pl.BlockSpec((2, _BM, 128), lambda i: (0, i, 0)),
        out_shape=jax.ShapeDtypeStruct((2, NP, 128), jnp.float32),
    )(h_cat, agg_cat, W1, b1, W2, b2)


def _readout_body(h_ref, a_ref, w1_ref, b1_ref, w2_ref, b2_ref,
                  wp_ref, bp_ref, out_ref, acc_ref):
    i = pl.program_id(0)
    hf = jnp.concatenate([h_ref[0] + a_ref[0], h_ref[1] + a_ref[1]], axis=-1)
    t = jnp.maximum(jnp.dot(hf, w1_ref[...],
                            preferred_element_type=jnp.float32) + b1_ref[...], 0.0)
    row = lax.broadcasted_iota(jnp.int32, (_BM, 1), 0) + i * _BM
    t = jnp.where(row < N, t, 0.0)
    part = jnp.sum(t, axis=0, keepdims=True)

    @pl.when(i == 0)
    def _():
        acc_ref[...] = jnp.zeros_like(acc_ref)

    acc_ref[...] += part

    @pl.when(i == _NBLK - 1)
    def _():
        pooled = acc_ref[...] * (1.0 / N)
        y = jnp.dot(pooled, w2_ref[...],
                    preferred_element_type=jnp.float32) + b2_ref[...]
        out_ref[...] = jnp.dot(y, wp_ref[...],
                               preferred_element_type=jnp.float32) + bp_ref[...]


def _readout(h_cat, agg_cat, W1, b1, W2, b2, W_p, b_p):
    full = lambda i: (0, 0)
    return pl.pallas_call(
        _readout_body,
        grid=(_NBLK,),
        in_specs=[
            pl.BlockSpec((2, _BM, 128), lambda i: (0, i, 0)),
            pl.BlockSpec((2, _BM, 128), lambda i: (0, i, 0)),
            pl.BlockSpec((256, 256), full),
            pl.BlockSpec((1, 256), full),
            pl.BlockSpec((256, 256), full),
            pl.BlockSpec((1, 256), full),
            pl.BlockSpec((256, 128), full),
            pl.BlockSpec((1, 128), full),
        ],
        out_specs=pl.BlockSpec((1, 128), full),
        out_shape=jax.ShapeDtypeStruct((1, 128), jnp.float32),
        scratch_shapes=[pltpu.VMEM((1, 256), jnp.float32)],
    )(h_cat, agg_cat, W1, b1, W2, b2, W_p, b_p)


def kernel(x, edge_index, W_in, W1_0, b1_0, W2_0, b2_0,
           W1_1, b1_1, W2_1, b2_1, W_p, b_p):
    x_pad = jnp.pad(x, ((0, NP - N), (0, 0)))
    src = jnp.pad(edge_index[0], (0, EP - E), constant_values=N)
    pad_dst = N + (jnp.arange(EP - E, dtype=jnp.int32) % (NP - N))
    dst = jnp.concatenate([edge_index[1], pad_dst])
    src2 = jnp.stack([src, src + NP]).reshape(2, 16, NCHUNK, CHUNK)
    dstp = dst.reshape(16, NCHUNK, CHUNK)
    srcx = src.reshape(2, 16, NCHUNK_X, CHUNK)
    dstx = dst.reshape(2, 16, NCHUNK_X, CHUNK)
    zblk = jnp.zeros((ROWS_PER_TILE, 128), jnp.float32)

    b1_0r = b1_0.reshape(1, 256)
    b2_0r = b2_0.reshape(1, 256)
    b1_1r = b1_1.reshape(1, 256)
    b2_1r = b2_1.reshape(1, 256)
    b_pr = b_p.reshape(1, 128)

    Wx1 = W_in @ W1_0
    aggx = _make_segsum_x()(x_pad, srcx, dstx, zblk)
    h_cat = _proj_in(x_pad, W_in)
    h1 = _gin0_layer(h_cat, aggx, W1_0, Wx1, b1_0r, W2_0, b2_0r)
    agg1 = _make_segsum()(h1.reshape(2 * NP, 128), src2, dstp, zblk)
    return _readout(h1, agg1, W1_1, b1_1r, W2_1, b2_1r, W_p, b_pr)


# speedup vs baseline: 1.0309x; 1.0309x over previous
import functools

import jax
import jax.numpy as jnp
from jax import lax
from jax.experimental import pallas as pl
from jax.experimental.pallas import tpu as pltpu
from jax.experimental.pallas import tpu_sc as plsc

N = 10000
NP = 10240
E = 320000
EP = 327680
CHUNK = 128
NCHUNK = EP // (16 * CHUNK)
GROUP = 16
GSEG = 32
ROWS_PER_TILE = NP // 16

@functools.lru_cache(maxsize=1)
def _make_segsum():
    mesh = plsc.VectorSubcoreMesh(
        core_axis_name="c", subcore_axis_name="s", num_cores=2, num_subcores=16)

    @functools.partial(
        pl.kernel,
        out_type=jax.ShapeDtypeStruct((2, NP, 128), jnp.float32),
        mesh=mesh,
        scratch_types=[
            pltpu.VMEM((GSEG, CHUNK), jnp.int32),
            pltpu.VMEM((GSEG, CHUNK), jnp.int32),
            pltpu.VMEM((CHUNK, 128), jnp.float32),
            pltpu.VMEM((CHUNK, 128), jnp.float32),
            pltpu.VMEM_SHARED((NP, 128), jnp.float32),
            pltpu.SemaphoreType.DMA,
            pltpu.SemaphoreType.DMA,
            pltpu.SemaphoreType.DMA,
            pltpu.SemaphoreType.DMA,
        ],
    )
    def _segsum(h_flat, src2, dstp, zblk, agg_out,
                src_v, dst_v, rows_a, rows_b, agg_sh, ga, gb, sa, sb):
        c = lax.axis_index("c")
        s = lax.axis_index("s")
        pltpu.sync_copy(zblk, agg_sh.at[pl.ds(s * ROWS_PER_TILE, ROWS_PER_TILE)])
        plsc.subcore_barrier()

        def gather(j, buf, sem):
            pltpu.async_copy(h_flat.at[src_v.at[j]], buf, sem)

        def gwait(buf, sem):
            pltpu.make_async_copy(h_flat.at[src_v.at[0]], buf, sem).wait()

        def scat(j, buf, sem):
            pltpu.async_copy(buf, agg_sh.at[dst_v.at[j]], sem, add=True)

        def swait(buf, sem):
            pltpu.make_async_copy(buf, agg_sh.at[dst_v.at[0]], sem).wait()

        NPAIR = GSEG // 2

        def group(g, carry):
            pltpu.sync_copy(src2.at[c, s, pl.ds(g * GSEG, GSEG)], src_v)
            pltpu.sync_copy(dstp.at[s, pl.ds(g * GSEG, GSEG)], dst_v)
            gather(0, rows_a, ga)

            def pair(p, carry2):
                j0 = 2 * p
                gwait(rows_a, ga)

                @pl.when(p > 0)
                def _():
                    swait(rows_b, sb)

                gather(j0 + 1, rows_b, gb)
                scat(j0, rows_a, sa)
                gwait(rows_b, gb)
                swait(rows_a, sa)

                @pl.when(p < NPAIR - 1)
                def _():
                    gather(j0 + 2, rows_a, ga)

                scat(j0 + 1, rows_b, sb)
                return carry2

            carry = lax.fori_loop(0, NPAIR, pair, carry)
            swait(rows_b, sb)
            return carry

        lax.fori_loop(0, NCHUNK // GSEG, group, 0)
        plsc.subcore_barrier()
        pltpu.sync_copy(agg_sh.at[pl.ds(s * ROWS_PER_TILE, ROWS_PER_TILE)],
                        agg_out.at[c, pl.ds(s * ROWS_PER_TILE, ROWS_PER_TILE)])

    return _segsum


NCHUNK_X = EP // (2 * 16 * CHUNK)


@functools.lru_cache(maxsize=1)
def _make_segsum_x():
    mesh = plsc.VectorSubcoreMesh(
        core_axis_name="c", subcore_axis_name="s", num_cores=2, num_subcores=16)

    @functools.partial(
        pl.kernel,
        out_type=jax.ShapeDtypeStruct((2, NP, 128), jnp.float32),
        mesh=mesh,
        scratch_types=[
            pltpu.VMEM((GROUP, CHUNK), jnp.int32),
            pltpu.VMEM((GROUP, CHUNK), jnp.int32),
            pltpu.VMEM((CHUNK, 128), jnp.float32),
            pltpu.VMEM((CHUNK, 128), jnp.float32),
            pltpu.VMEM_SHARED((NP, 128), jnp.float32),
            pltpu.SemaphoreType.DMA,
            pltpu.SemaphoreType.DMA,
            pltpu.SemaphoreType.DMA,
            pltpu.SemaphoreType.DMA,
        ],
    )
    def _segsum_x(x_tab, srcx, dstx, zblk, acc_out,
                  src_v, dst_v, rows_a, rows_b, agg_sh, ga, gb, sa, sb):
        c = lax.axis_index("c")
        s = lax.axis_index("s")
        pltpu.sync_copy(zblk, agg_sh.at[pl.ds(s * ROWS_PER_TILE, ROWS_PER_TILE)])
        plsc.subcore_barrier()

        def gather(j, buf, sem):
            pltpu.async_copy(x_tab.at[src_v.at[j]], buf, sem)

        def gwait(buf, sem):
            pltpu.make_async_copy(x_tab.at[src_v.at[0]], buf, sem).wait()

        def scat(j, buf, sem):
            pltpu.async_copy(buf, agg_sh.at[dst_v.at[j]], sem, add=True)

        def swait(buf, sem):
            pltpu.make_async_copy(buf, agg_sh.at[dst_v.at[0]], sem).wait()

        NPAIR = GROUP // 2

        def group(g, carry):
            pltpu.sync_copy(srcx.at[c, s, pl.ds(g * GROUP, GROUP)], src_v)
            pltpu.sync_copy(dstx.at[c, s, pl.ds(g * GROUP, GROUP)], dst_v)
            gather(0, rows_a, ga)

            def pair(p, carry2):
                j0 = 2 * p
                gwait(rows_a, ga)

                @pl.when(p > 0)
                def _():
                    swait(rows_b, sb)

                gather(j0 + 1, rows_b, gb)
                scat(j0, rows_a, sa)
                gwait(rows_b, gb)
                swait(rows_a, sa)

                @pl.when(p < NPAIR - 1)
                def _():
                    gather(j0 + 2, rows_a, ga)

                scat(j0 + 1, rows_b, sb)
                return carry2

            carry = lax.fori_loop(0, NPAIR, pair, carry)
            swait(rows_b, sb)
            return carry

        lax.fori_loop(0, NCHUNK_X // GROUP, group, 0)
        plsc.subcore_barrier()
        pltpu.sync_copy(agg_sh.at[pl.ds(s * ROWS_PER_TILE, ROWS_PER_TILE)],
                        acc_out.at[c, pl.ds(s * ROWS_PER_TILE, ROWS_PER_TILE)])

    return _segsum_x


_BM = 512
_NBLK = NP // _BM


def _proj_body(x_ref, w_ref, out_ref):
    out_ref[0] = jnp.dot(x_ref[...], w_ref[...],
                         preferred_element_type=jnp.float32)


def _proj_in(x_pad, W_in):
    return pl.pallas_call(
        _proj_body,
        grid=(2, _NBLK),
        in_specs=[
            pl.BlockSpec((_BM, 128), lambda c, i: (i, 0)),
            pl.BlockSpec((128, 128), lambda c, i: (0, c)),
        ],
        out_specs=pl.BlockSpec((1, _BM, 128), lambda c, i: (c, i, 0)),
        out_shape=jax.ShapeDtypeStruct((2, NP, 128), jnp.float32),
    )(x_pad, W_in)


def _gin0_body(h_ref, a_ref, w1_ref, wx1_ref, b1_ref, w2_ref, b2_ref, out_ref):
    hf = jnp.concatenate([h_ref[0], h_ref[1]], axis=-1)
    ax = a_ref[0] + a_ref[1]
    u = jnp.maximum(
        jnp.dot(hf, w1_ref[...], preferred_element_type=jnp.float32)
        + jnp.dot(ax, wx1_ref[...], preferred_element_type=jnp.float32)
        + b1_ref[...], 0.0)
    h2 = jnp.dot(u, w2_ref[...], preferred_element_type=jnp.float32) + b2_ref[...]
    out_ref[0] = h2[:, :128]
    out_ref[1] = h2[:, 128:]


def _gin0_layer(h_cat, ax_pair, W1, Wx1, b1, W2, b2):
    full = lambda i: (0, 0)
    return pl.pallas_call(
        _gin0_body,
        grid=(_NBLK,),
        in_specs=[
            pl.BlockSpec((2, _BM, 128), lambda i: (0, i, 0)),
            pl.BlockSpec((2, _BM, 128), lambda i: (0, i, 0)),
            pl.BlockSpec((256, 256), full),
            pl.BlockSpec((128, 256), full),
            pl.BlockSpec((1, 256), full),
            pl.BlockSpec((256, 256), full),
            pl.BlockSpec((1, 256), full),
        ],
        out_specs=pl.BlockSpec((2, _BM, 128), lambda i: (0, i, 0)),
        out_shape=jax.ShapeDtypeStruct((2, NP, 128), jnp.float32),
    )(h_cat, ax_pair, W1, Wx1, b1, W2, b2)


def _gin_body(h_ref, a_ref, w1_ref, b1_ref, w2_ref, b2_ref, out_ref):
    hf = jnp.concatenate([h_ref[0] + a_ref[0], h_ref[1] + a_ref[1]], axis=-1)
    u = jnp.maximum(jnp.dot(hf, w1_ref[...],
                            preferred_element_type=jnp.float32) + b1_ref[...], 0.0)
    h2 = jnp.dot(u, w2_ref[...], preferred_element_type=jnp.float32) + b2_ref[...]
    out_ref[0] = h2[:, :128]
    out_ref[1] = h2[:, 128:]


def _gin_layer(h_cat, agg_cat, W1, b1, W2, b2):
    full = lambda i: (0, 0)
    return pl.pallas_call(
        _gin_body,
        grid=(_NBLK,),
        in_specs=[
            pl.BlockSpec((2, _BM, 128), lambda i: (0, i, 0)),
            pl.BlockSpec((2, _BM, 128), lambda i: (0, i, 0)),
            pl.BlockSpec((256, 256), full),
            pl.BlockSpec((1, 256), full),
            pl.BlockSpec((256, 256), full),
            pl.BlockSpec((1, 256), full),
        ],
        out_specs=pl.BlockSpec((2, _BM, 128), lambda i: (0, i, 0)),
        out_shape=jax.ShapeDtypeStruct((2, NP, 128), jnp.float32),
    )(h_cat, agg_cat, W1, b1, W2, b2)


def _readout_body(h_ref, a_ref, w1_ref, b1_ref, w2_ref, b2_ref,
                  wp_ref, bp_ref, out_ref, acc_ref):
    i = pl.program_id(0)
    hf = jnp.concatenate([h_ref[0] + a_ref[0], h_ref[1] + a_ref[1]], axis=-1)
    t = jnp.maximum(jnp.dot(hf, w1_ref[...],
                            preferred_element_type=jnp.float32) + b1_ref[...], 0.0)
    row = lax.broadcasted_iota(jnp.int32, (_BM, 1), 0) + i * _BM
    t = jnp.where(row < N, t, 0.0)
    part = jnp.sum(t, axis=0, keepdims=True)

    @pl.when(i == 0)
    def _():
        acc_ref[...] = jnp.zeros_like(acc_ref)

    acc_ref[...] += part

    @pl.when(i == _NBLK - 1)
    def _():
        pooled = acc_ref[...] * (1.0 / N)
        y = jnp.dot(pooled, w2_ref[...],
                    preferred_element_type=jnp.float32) + b2_ref[...]
        out_ref[...] = jnp.dot(y, wp_ref[...],
                               preferred_element_type=jnp.float32) + bp_ref[...]


def _readout(h_cat, agg_cat, W1, b1, W2, b2, W_p, b_p):
    full = lambda i: (0, 0)
    return pl.pallas_call(
        _readout_body,
        grid=(_NBLK,),
        in_specs=[
            pl.BlockSpec((2, _BM, 128), lambda i: (0, i, 0)),
            pl.BlockSpec((2, _BM, 128), lambda i: (0, i, 0)),
            pl.BlockSpec((256, 256), full),
            pl.BlockSpec((1, 256), full),
            pl.BlockSpec((256, 256), full),
            pl.BlockSpec((1, 256), full),
            pl.BlockSpec((256, 128), full),
            pl.BlockSpec((1, 128), full),
        ],
        out_specs=pl.BlockSpec((1, 128), full),
        out_shape=jax.ShapeDtypeStruct((1, 128), jnp.float32),
        scratch_shapes=[pltpu.VMEM((1, 256), jnp.float32)],
    )(h_cat, agg_cat, W1, b1, W2, b2, W_p, b_p)


def kernel(x, edge_index, W_in, W1_0, b1_0, W2_0, b2_0,
           W1_1, b1_1, W2_1, b2_1, W_p, b_p):
    x_pad = jnp.pad(x, ((0, NP - N), (0, 0)))
    src = jnp.pad(edge_index[0], (0, EP - E), constant_values=N)
    dst = jnp.pad(edge_index[1], (0, EP - E), constant_values=N)
    src2 = jnp.stack([src, src + NP]).reshape(2, 16, NCHUNK, CHUNK)
    dstp = dst.reshape(16, NCHUNK, CHUNK)
    srcx = src.reshape(2, 16, NCHUNK_X, CHUNK)
    dstx = dst.reshape(2, 16, NCHUNK_X, CHUNK)
    zblk = jnp.zeros((ROWS_PER_TILE, 128), jnp.float32)

    b1_0r = b1_0.reshape(1, 256)
    b2_0r = b2_0.reshape(1, 256)
    b1_1r = b1_1.reshape(1, 256)
    b2_1r = b2_1.reshape(1, 256)
    b_pr = b_p.reshape(1, 128)

    Wx1 = W_in @ W1_0
    aggx = _make_segsum_x()(x_pad, srcx, dstx, zblk)
    h_cat = _proj_in(x_pad, W_in)
    h1 = _gin0_layer(h_cat, aggx, W1_0, Wx1, b1_0r, W2_0, b2_0r)
    agg1 = _make_segsum()(h1.reshape(2 * NP, 128), src2, dstp, zblk)
    return _readout(h1, agg1, W1_1, b1_1r, W2_1, b2_1r, W_p, b_pr)
